# Initial kernel scaffold; baseline (speedup 1.0000x reference)
import jax
import jax.numpy as jnp
from jax.experimental import pallas as pl


def kernel(x, adj_batch, W_hidden, b_hidden):
    raise NotImplementedError("write your pallas kernel here")


# speedup vs baseline: 1.4335x; 1.4335x over previous
import functools

import jax
import jax.numpy as jnp
from jax import lax
from jax.experimental import pallas as pl
from jax.experimental.pallas import tpu as pltpu
from jax.experimental.pallas import tpu_sc as plsc

LANES = 16
BLK = 8


def _lane_splat(v, lane):
    idx = jnp.full((LANES, 1), lane, dtype=jnp.int32)
    dnums = lax.GatherDimensionNumbers(
        offset_dims=(), collapsed_slice_dims=(0,), start_index_map=(0,)
    )
    return lax.gather(
        v, idx, dnums, slice_sizes=(1,),
        mode=lax.GatherScatterMode.PROMISE_IN_BOUNDS,
    )


def _sc_gather_sum(x2d, idx3, scr2, nw, nodes_per_w, nblk, nas, f):
    np_total = nw * nodes_per_w
    fregs = f // LANES
    mesh = plsc.VectorSubcoreMesh(
        core_axis_name="c", subcore_axis_name="s", num_cores=2, num_subcores=16
    )

    @functools.partial(
        pl.kernel,
        out_type=jax.ShapeDtypeStruct((np_total, f), jnp.float32),
        mesh=mesh,
        scratch_types=[
            pltpu.VMEM((nblk, BLK * nas), jnp.int32),
            pltpu.VMEM((nodes_per_w * nas + LANES,), jnp.float32),
            pltpu.VMEM((BLK * nas, f), jnp.float32),
            pltpu.VMEM((nodes_per_w, f), jnp.float32),
            pltpu.SemaphoreType.DMA,
        ],
    )
    def k(x_hbm, idx_hbm, scr_hbm, out_hbm, idx_v, scr_v, rows_v, g_v, sem):
        wid = lax.axis_index("s") * 2 + lax.axis_index("c")
        node_base = wid * nodes_per_w
        pltpu.sync_copy(idx_hbm.at[wid], idx_v)
        pltpu.sync_copy(scr_hbm.at[wid], scr_v.at[pl.ds(0, nodes_per_w * nas)])

        @pl.loop(0, nblk)
        def _block(t):
            pltpu.async_copy(x_hbm.at[idx_v.at[t]], rows_v, sem).wait()
            sbase = t * (BLK * nas)
            for i in range(BLK):
                accs = [jnp.zeros((LANES,), jnp.float32) for _ in range(fregs)]
                s16 = scr_v[pl.ds(sbase + i * nas, LANES)]
                for a in range(nas):
                    sv = _lane_splat(s16, a)
                    r = i * nas + a
                    for j in range(fregs):
                        accs[j] = accs[j] + sv * rows_v[r, pl.ds(j * LANES, LANES)]
                node = t * BLK + i
                for j in range(fregs):
                    g_v[node, pl.ds(j * LANES, LANES)] = accs[j]

        pltpu.sync_copy(g_v, out_hbm.at[pl.ds(node_base, nodes_per_w)])

    return k(x2d, idx3, scr2)


def _mlp_body(g_ref, x_ref, w1_ref, w2_ref, b_ref, o_ref):
    o_ref[...] = (
        jnp.dot(g_ref[...], w1_ref[...], preferred_element_type=jnp.float32)
        + jnp.dot(x_ref[...], w2_ref[...], preferred_element_type=jnp.float32)
        + b_ref[...]
    )


def _tc_mlp(g, x2d, w1, w2, b2, n, f, out_dim, m_blk):
    grid = (n // m_blk,)
    return pl.pallas_call(
        _mlp_body,
        grid=grid,
        in_specs=[
            pl.BlockSpec((m_blk, f), lambda i: (i, 0)),
            pl.BlockSpec((m_blk, f), lambda i: (i, 0)),
            pl.BlockSpec((f, out_dim), lambda i: (0, 0)),
            pl.BlockSpec((f, out_dim), lambda i: (0, 0)),
            pl.BlockSpec((1, out_dim), lambda i: (0, 0)),
        ],
        out_specs=pl.BlockSpec((m_blk, out_dim), lambda i: (i, 0)),
        out_shape=jax.ShapeDtypeStruct((n, out_dim), jnp.float32),
    )(g, x2d, w1, w2, b2)


def kernel(x, adj_batch, W_hidden, b_hidden):
    n, b, f = x.shape
    nas = adj_batch.shape[-1]
    out_dim = W_hidden.shape[0]

    x2d = x.reshape(n, f)
    ab = adj_batch[0]
    scores = ab[0]
    idx = ab[1].astype(jnp.int32)

    nw = 32
    nodes_per_w = -(-n // (nw * BLK)) * BLK
    nblk = nodes_per_w // BLK
    np_total = nw * nodes_per_w

    idx_p = jnp.zeros((np_total, nas), jnp.int32).at[:n].set(idx)
    scr_p = jnp.zeros((np_total, nas), jnp.float32).at[:n].set(scores)
    idx3 = idx_p.reshape(nw, nblk, BLK * nas)
    scr2 = scr_p.reshape(nw, nodes_per_w * nas)

    g = _sc_gather_sum(x2d, idx3, scr2, nw, nodes_per_w, nblk, nas, f)[:n]

    w1 = W_hidden[:, :f].T * (1.0 / nas)
    w2 = W_hidden[:, f:].T
    b2 = b_hidden.reshape(1, out_dim)

    m_blk = 1000 if n % 1000 == 0 else 8
    out2 = _tc_mlp(g, x2d, w1, w2, b2, n, f, out_dim, m_blk)
    return out2.reshape(n, b, out_dim)


# speedup vs baseline: 1.6005x; 1.1165x over previous
import functools

import jax
import jax.numpy as jnp
from jax import lax
from jax.experimental import pallas as pl
from jax.experimental.pallas import tpu as pltpu
from jax.experimental.pallas import tpu_sc as plsc

LANES = 16
BLK = 8


def _lane_splat(v, lane):
    idx = jnp.full((LANES, 1), lane, dtype=jnp.int32)
    dnums = lax.GatherDimensionNumbers(
        offset_dims=(), collapsed_slice_dims=(0,), start_index_map=(0,)
    )
    return lax.gather(
        v, idx, dnums, slice_sizes=(1,),
        mode=lax.GatherScatterMode.PROMISE_IN_BOUNDS,
    )


def _sc_gather_sum(x2d, idx3, scr2, nw, nodes_per_w, nblk, nas, f):
    np_total = nw * nodes_per_w
    fregs = f // LANES
    mesh = plsc.VectorSubcoreMesh(
        core_axis_name="c", subcore_axis_name="s", num_cores=2, num_subcores=16
    )

    nbuf = 4

    @functools.partial(
        pl.kernel,
        out_type=jax.ShapeDtypeStruct((np_total, f), jnp.float32),
        mesh=mesh,
        scratch_types=[
            pltpu.VMEM((nblk, BLK * nas), jnp.int32),
            pltpu.VMEM((nodes_per_w * nas + LANES,), jnp.float32),
            [pltpu.VMEM((BLK * nas, f), jnp.float32)] * nbuf,
            pltpu.VMEM((nodes_per_w, f), jnp.float32),
            [pltpu.SemaphoreType.DMA] * nbuf,
        ],
    )
    def k(x_hbm, idx_hbm, scr_hbm, out_hbm, idx_v, scr_v, rows, g_v, sems):
        wid = lax.axis_index("s") * 2 + lax.axis_index("c")
        node_base = wid * nodes_per_w
        pltpu.sync_copy(idx_hbm.at[wid], idx_v)
        pltpu.sync_copy(scr_hbm.at[wid], scr_v.at[pl.ds(0, nodes_per_w * nas)])

        for r in range(nbuf - 1):
            pltpu.async_copy(x_hbm.at[idx_v.at[r]], rows[r], sems[r])

        @pl.loop(0, nblk, step=nbuf)
        def _block(t):
            for r in range(nbuf):
                tb = t + r
                nxt = tb + (nbuf - 1)
                rn = (r + nbuf - 1) % nbuf

                @pl.when(nxt < nblk)
                def _():
                    pltpu.async_copy(x_hbm.at[idx_v.at[nxt]], rows[rn], sems[rn])

                pltpu.make_async_copy(x_hbm.at[idx_v.at[tb]], rows[r], sems[r]).wait()
                rows_v = rows[r]
                sbase = tb * (BLK * nas)
                for i in range(BLK):
                    accs = [jnp.zeros((LANES,), jnp.float32) for _ in range(fregs)]
                    s16 = scr_v[pl.ds(sbase + i * nas, LANES)]
                    for a in range(nas):
                        sv = _lane_splat(s16, a)
                        rr = i * nas + a
                        for j in range(fregs):
                            accs[j] = accs[j] + sv * rows_v[rr, pl.ds(j * LANES, LANES)]
                    node = tb * BLK + i
                    for j in range(fregs):
                        g_v[node, pl.ds(j * LANES, LANES)] = accs[j]

        pltpu.sync_copy(g_v, out_hbm.at[pl.ds(node_base, nodes_per_w)])

    return k(x2d, idx3, scr2)


def _mlp_body(g_ref, x_ref, w1_ref, w2_ref, b_ref, o_ref):
    o_ref[...] = (
        jnp.dot(g_ref[...], w1_ref[...], preferred_element_type=jnp.float32)
        + jnp.dot(x_ref[...], w2_ref[...], preferred_element_type=jnp.float32)
        + b_ref[...]
    )


def _tc_mlp(g, x2d, w1, w2, b2, n, f, out_dim, m_blk):
    grid = (n // m_blk,)
    return pl.pallas_call(
        _mlp_body,
        grid=grid,
        in_specs=[
            pl.BlockSpec((m_blk, f), lambda i: (i, 0)),
            pl.BlockSpec((m_blk, f), lambda i: (i, 0)),
            pl.BlockSpec((f, out_dim), lambda i: (0, 0)),
            pl.BlockSpec((f, out_dim), lambda i: (0, 0)),
            pl.BlockSpec((1, out_dim), lambda i: (0, 0)),
        ],
        out_specs=pl.BlockSpec((m_blk, out_dim), lambda i: (i, 0)),
        out_shape=jax.ShapeDtypeStruct((n, out_dim), jnp.float32),
    )(g, x2d, w1, w2, b2)


def kernel(x, adj_batch, W_hidden, b_hidden):
    n, b, f = x.shape
    nas = adj_batch.shape[-1]
    out_dim = W_hidden.shape[0]

    x2d = x.reshape(n, f)
    ab = adj_batch[0]
    scores = ab[0]
    idx = ab[1].astype(jnp.int32)

    nw = 32
    nodes_per_w = -(-n // (nw * BLK * 4)) * (BLK * 4)
    nblk = nodes_per_w // BLK
    np_total = nw * nodes_per_w

    idx_p = jnp.zeros((np_total, nas), jnp.int32).at[:n].set(idx)
    scr_p = jnp.zeros((np_total, nas), jnp.float32).at[:n].set(scores)
    idx3 = idx_p.reshape(nw, nblk, BLK * nas)
    scr2 = scr_p.reshape(nw, nodes_per_w * nas)

    g = _sc_gather_sum(x2d, idx3, scr2, nw, nodes_per_w, nblk, nas, f)

    w1 = W_hidden[:, :f].T * (1.0 / nas)
    w2 = W_hidden[:, f:].T
    b2 = b_hidden.reshape(1, out_dim)

    m_blk = 1000 if n % 1000 == 0 else 8
    out2 = _tc_mlp(g, x2d, w1, w2, b2, n, f, out_dim, m_blk)
    return out2.reshape(n, b, out_dim)


# speedup vs baseline: 2.6952x; 1.6840x over previous
import functools

import numpy as np

import jax
import jax.numpy as jnp
from jax import lax
from jax.experimental import pallas as pl
from jax.experimental.pallas import tpu as pltpu
from jax.experimental.pallas import tpu_sc as plsc

LANES = 16
BLK = 8


def _lane_splat(v, lane):
    idx = jnp.full((LANES, 1), lane, dtype=jnp.int32)
    dnums = lax.GatherDimensionNumbers(
        offset_dims=(), collapsed_slice_dims=(0,), start_index_map=(0,)
    )
    return lax.gather(
        v, idx, dnums, slice_sizes=(1,),
        mode=lax.GatherScatterMode.PROMISE_IN_BOUNDS,
    )


def _sc_gather_sum(x_i32, adj0, nw, nodes_per_w, nblk, nas, f):
    n_rows = x_i32.shape[0]
    fregs = f // LANES
    mesh = plsc.VectorSubcoreMesh(
        core_axis_name="c", subcore_axis_name="s", num_cores=2, num_subcores=16
    )

    nbuf = 4

    @functools.partial(
        pl.kernel,
        out_type=jax.ShapeDtypeStruct((n_rows, f), jnp.float32),
        mesh=mesh,
        compiler_params=pltpu.CompilerParams(use_tc_tiling_on_sc=False),
        scratch_types=[
            pltpu.VMEM((nodes_per_w, nas), jnp.float32),
            pltpu.VMEM((nodes_per_w * nas,), jnp.int32),
            pltpu.VMEM((nodes_per_w, nas), jnp.float32),
            [pltpu.VMEM((BLK * nas, f // 2), jnp.int32)] * nbuf,
            pltpu.VMEM((nodes_per_w, f), jnp.float32),
            pltpu.VMEM_SHARED((n_rows, f // 2), jnp.int32),
            [pltpu.SemaphoreType.DMA] * nbuf,
        ],
    )
    def k(x_hbm, adj_hbm, out_hbm, idx_vf, idx_v, scr_v, rows, g_v, x_sh, sems):
        wid = lax.axis_index("s") * 2 + lax.axis_index("c")
        sid = lax.axis_index("s")
        node_base = jnp.minimum(wid * nodes_per_w, n_rows - nodes_per_w)
        rows_per_tile = n_rows // 16
        pltpu.sync_copy(
            x_hbm.at[pl.ds(sid * rows_per_tile, rows_per_tile)],
            x_sh.at[pl.ds(sid * rows_per_tile, rows_per_tile)],
        )
        pltpu.sync_copy(adj_hbm.at[0, pl.ds(node_base, nodes_per_w)], scr_v)
        pltpu.sync_copy(adj_hbm.at[1, pl.ds(node_base, nodes_per_w)], idx_vf)

        @pl.loop(0, nodes_per_w)
        def _cvt(nd):
            idx_v[pl.ds(nd * nas, LANES)] = idx_vf[nd, :].astype(jnp.int32)

        plsc.subcore_barrier()

        for r in range(nbuf - 1):
            pltpu.async_copy(
                x_sh.at[idx_v.at[pl.ds(r * BLK * nas, BLK * nas)]], rows[r], sems[r]
            )

        @pl.loop(0, nblk, step=nbuf)
        def _block(t):
            for r in range(nbuf):
                tb = t + r
                nxt = tb + (nbuf - 1)
                rn = (r + nbuf - 1) % nbuf

                @pl.when(nxt < nblk)
                def _():
                    pltpu.async_copy(
                        x_sh.at[idx_v.at[pl.ds(nxt * BLK * nas, BLK * nas)]],
                        rows[rn], sems[rn],
                    )

                pltpu.make_async_copy(
                    x_sh.at[idx_v.at[pl.ds(tb * BLK * nas, BLK * nas)]],
                    rows[r], sems[r],
                ).wait()
                rows_v = rows[r]
                for i in range(BLK):
                    accs = [jnp.zeros((LANES,), jnp.float32) for _ in range(fregs)]
                    node = tb * BLK + i
                    s16 = scr_v[node, :]
                    for a in range(nas):
                        sv = _lane_splat(s16, a)
                        rr = i * nas + a
                        for j in range(fregs // 2):
                            w = rows_v[rr, pl.ds(j * LANES, LANES)]
                            ev = lax.bitcast_convert_type(
                                lax.shift_left(w, 16), jnp.float32
                            )
                            od = lax.bitcast_convert_type(
                                lax.bitwise_and(w, jnp.int32(-65536)), jnp.float32
                            )
                            accs[2 * j] = accs[2 * j] + sv * ev
                            accs[2 * j + 1] = accs[2 * j + 1] + sv * od
                    for j in range(fregs):
                        g_v[node, pl.ds(j * LANES, LANES)] = accs[j]

        pltpu.sync_copy(g_v, out_hbm.at[pl.ds(node_base, nodes_per_w)])

    return k(x_i32, adj0)


def _mlp_body(g_ref, x_ref, w1_ref, w2_ref, b_ref, o_ref):
    o_ref[...] = (
        jnp.dot(g_ref[...], w1_ref[...], preferred_element_type=jnp.float32)
        + jnp.dot(x_ref[...], w2_ref[...], preferred_element_type=jnp.float32)
        + b_ref[...]
    )


def _tc_mlp(g, x2d, w1, w2, b2, n, f, out_dim, m_blk):
    grid = (n // m_blk,)
    return pl.pallas_call(
        _mlp_body,
        grid=grid,
        in_specs=[
            pl.BlockSpec((m_blk, f), lambda i: (i, 0)),
            pl.BlockSpec((m_blk, f), lambda i: (i, 0)),
            pl.BlockSpec((f, out_dim), lambda i: (0, 0)),
            pl.BlockSpec((f, out_dim), lambda i: (0, 0)),
            pl.BlockSpec((1, out_dim), lambda i: (0, 0)),
        ],
        out_specs=pl.BlockSpec((m_blk, out_dim), lambda i: (i, 0)),
        out_shape=jax.ShapeDtypeStruct((n, out_dim), jnp.float32),
    )(g, x2d, w1, w2, b2)


def kernel(x, adj_batch, W_hidden, b_hidden):
    n, b, f = x.shape
    nas = adj_batch.shape[-1]
    out_dim = W_hidden.shape[0]

    x2d = x.reshape(n, f)

    nw = 32
    nodes_per_w = -(-n // (nw * BLK * 4)) * (BLK * 4)
    nblk = nodes_per_w // BLK

    x_bf = x2d.astype(jnp.bfloat16)
    x_i32 = lax.bitcast_convert_type(x_bf.reshape(n, f // 2, 2), jnp.int32)
    g = _sc_gather_sum(x_i32, adj_batch[0], nw, nodes_per_w, nblk, nas, f)

    w1 = W_hidden[:, :f].T * (1.0 / nas)
    perm = np.concatenate(
        [np.concatenate([np.arange(c, c + 32, 2), np.arange(c + 1, c + 32, 2)])
         for c in range(0, f, 32)]
    )
    w1 = w1[perm, :]
    w2 = W_hidden[:, f:].T
    b2 = b_hidden.reshape(1, out_dim)

    m_blk = 1000 if n % 1000 == 0 else 8
    out2 = _tc_mlp(g, x2d, w1, w2, b2, n, f, out_dim, m_blk)
    return out2.reshape(n, b, out_dim)
